# baseline (speedup 1.0000x reference)
import jax
import jax.numpy as jnp
from jax.experimental import pallas as pl
from jax.experimental.pallas import tpu as pltpu

_BN_EPS = 1e-5
_N_OUT = 2

_L = (
    (3, 3, 16, 74, 72, 36),
    (3, 16, 32, 38, 36, 18),
    (3, 32, 64, 20, 18, 9),
    (2, 64, 128, 11, 10, 5),
    (2, 128, 32, 7, 6, 3),
)


def _rows(hp, ho):
    return (ho - 1) * hp + ho


def _body(p_ref, w1_ref, b1_ref, w2_ref, b2_ref, w3_ref, b3_ref,
          w4_ref, b4_ref, w5_ref, b5_ref, wfa_ref, wfb_ref, fcb_ref,
          out_ref, act1, in2, act2, in3, act3, in4, act4, in5, act5, pool5):
    bf16 = jnp.bfloat16

    def leaky(v):
        return jnp.where(v > 0, v, 0.01 * v)

    def conv(in_ref, w_ref, b_ref, act_ref, k, hp, r):
        acc = None
        for di in range(k):
            for dj in range(k):
                part = jnp.dot(in_ref[pl.ds(di * hp + dj, r), :],
                               w_ref[di * k + dj],
                               preferred_element_type=jnp.float32)
                acc = part if acc is None else acc + part
        act_ref[...] = leaky(acc + b_ref[...])

    def pool2x2(act_ref, sp, ho2, i2):
        s = 2 * i2 * sp
        a00 = act_ref[pl.ds(s, ho2, stride=2), :]
        a01 = act_ref[pl.ds(s + 1, ho2, stride=2), :]
        a10 = act_ref[pl.ds(s + sp, ho2, stride=2), :]
        a11 = act_ref[pl.ds(s + sp + 1, ho2, stride=2), :]
        return jnp.maximum(jnp.maximum(a00, a01), jnp.maximum(a10, a11))

    def pool_pad(act_ref, sp, ho, ho2, dst_ref, dst_hp):
        zrow = jnp.zeros((1, dst_ref.shape[1]), bf16)
        for i2 in range(ho2):
            hm = pool2x2(act_ref, sp, ho2, i2).astype(bf16)
            row = jnp.concatenate([zrow, hm, zrow], axis=0)
            dst_ref[pl.ds((i2 + 1) * dst_hp, dst_hp), :] = row

    def zero_tb(dst_ref, hp):
        z = jnp.zeros((hp, dst_ref.shape[1]), bf16)
        dst_ref[pl.ds(0, hp), :] = z
        dst_ref[pl.ds((hp - 1) * hp, hp), :] = z

    hp1, ho1 = _L[0][3], _L[0][4]
    r1 = _rows(hp1, ho1)
    acc1 = None
    for di in range(3):
        for dj in range(3):
            part = jnp.dot(p_ref[0, pl.ds(di * hp1 + dj, r1), :],
                           w1_ref[di * 3 + dj],
                           preferred_element_type=jnp.float32)
            acc1 = part if acc1 is None else acc1 + part
    act1[...] = leaky(acc1 + b1_ref[...])

    ins = (None, in2, in3, in4, in5)
    acts = (act1, act2, act3, act4, act5)
    ws = (None, w2_ref, w3_ref, w4_ref, w5_ref)
    bs = (None, b2_ref, b3_ref, b4_ref, b5_ref)
    sps = tuple(l[3] for l in _L)

    for i in range(5):
        k, _, _, hp, ho, ho2 = _L[i]
        if i > 0:
            conv(ins[i], ws[i], bs[i], acts[i], k, hp, _rows(hp, ho))
        if i < 4:
            nhp = _L[i + 1][3]
            zero_tb(ins[i + 1], nhp)
            pool_pad(acts[i], sps[i], ho, ho2, ins[i + 1], nhp)
        else:
            for i2 in range(ho2):
                pool5[pl.ds(i2 * ho2, ho2), :] = (
                    pool2x2(acts[i], sps[i], ho2, i2).astype(bf16))

    v = pool5[...].astype(jnp.float32)
    s0 = jnp.sum(v * wfa_ref[...])
    s1 = jnp.sum(v * wfb_ref[...])
    idx = jax.lax.broadcasted_iota(jnp.int32, (1, _N_OUT), 1)
    logits = fcb_ref[...] + jnp.where(idx == 0, s0, s1)
    out_ref[...] = jax.nn.sigmoid(logits).reshape(1, 1, _N_OUT)


def kernel(c1_w, c1_b, c1_g, c1_beta, c1_m, c1_v,
           c2_w, c2_b, c2_g, c2_beta, c2_m, c2_v,
           c3_w, c3_b, c3_g, c3_beta, c3_m, c3_v,
           c4_w, c4_b, c4_g, c4_beta, c4_m, c4_v,
           c5_w, c5_b, c5_g, c5_beta, c5_m, c5_v,
           fc_w, fc_b, x):
    B = x.shape[0]
    bf16 = jnp.bfloat16

    def fold(w, b, g, beta, m, v):
        s = g * jax.lax.rsqrt(v + _BN_EPS)
        return w * s, ((b - m) * s + beta).reshape(1, -1)

    fw1, fb1 = fold(c1_w, c1_b, c1_g, c1_beta, c1_m, c1_v)
    fw2, fb2 = fold(c2_w, c2_b, c2_g, c2_beta, c2_m, c2_v)
    fw3, fb3 = fold(c3_w, c3_b, c3_g, c3_beta, c3_m, c3_v)
    fw4, fb4 = fold(c4_w, c4_b, c4_g, c4_beta, c4_m, c4_v)
    fw5, fb5 = fold(c5_w, c5_b, c5_g, c5_beta, c5_m, c5_v)

    hp1 = _L[0][3]
    xh = jnp.transpose(x, (0, 2, 3, 1)).astype(bf16)
    xp = jnp.pad(xh, ((0, 0), (1, 1), (1, 1), (0, 0)))
    xflat = xp.reshape(B, hp1 * hp1, _L[0][1])

    w1s = fw1.reshape(9, _L[0][1], _L[0][2]).astype(bf16)
    w2s = fw2.reshape(9, _L[1][1], _L[1][2]).astype(bf16)
    w3s = fw3.reshape(9, _L[2][1], _L[2][2]).astype(bf16)
    w4s = fw4.reshape(4, _L[3][1], _L[3][2]).astype(bf16)
    w5s = fw5.reshape(4, _L[4][1], _L[4][2]).astype(bf16)

    fs = _L[4][5] ** 2
    wf = fc_w.reshape(_L[4][2], fs, _N_OUT)
    wfa = wf[:, :, 0].T
    wfb = wf[:, :, 1].T
    fcb = fc_b.reshape(1, _N_OUT)

    const2 = lambda shape: pl.BlockSpec(shape, lambda b: (0, 0))
    const3 = lambda shape: pl.BlockSpec(shape, lambda b: (0, 0, 0))

    out = pl.pallas_call(
        _body,
        out_shape=jax.ShapeDtypeStruct((B, 1, _N_OUT), jnp.float32),
        grid=(B,),
        in_specs=[
            pl.BlockSpec((1, hp1 * hp1, _L[0][1]), lambda b: (b, 0, 0)),
            const3(w1s.shape), const2(fb1.shape),
            const3(w2s.shape), const2(fb2.shape),
            const3(w3s.shape), const2(fb3.shape),
            const3(w4s.shape), const2(fb4.shape),
            const3(w5s.shape), const2(fb5.shape),
            const2(wfa.shape), const2(wfb.shape), const2(fcb.shape),
        ],
        out_specs=pl.BlockSpec((1, 1, _N_OUT), lambda b: (b, 0, 0)),
        scratch_shapes=[
            pltpu.VMEM((_rows(_L[0][3], _L[0][4]), _L[0][2]), jnp.float32),
            pltpu.VMEM((_L[1][3] * _L[1][3], _L[1][1]), bf16),
            pltpu.VMEM((_rows(_L[1][3], _L[1][4]), _L[1][2]), jnp.float32),
            pltpu.VMEM((_L[2][3] * _L[2][3], _L[2][1]), bf16),
            pltpu.VMEM((_rows(_L[2][3], _L[2][4]), _L[2][2]), jnp.float32),
            pltpu.VMEM((_L[3][3] * _L[3][3], _L[3][1]), bf16),
            pltpu.VMEM((_rows(_L[3][3], _L[3][4]), _L[3][2]), jnp.float32),
            pltpu.VMEM((_L[4][3] * _L[4][3], _L[4][1]), bf16),
            pltpu.VMEM((_rows(_L[4][3], _L[4][4]), _L[4][2]), jnp.float32),
            pltpu.VMEM((fs, _L[4][2]), bf16),
        ],
        compiler_params=pltpu.CompilerParams(
            dimension_semantics=("parallel",),
            vmem_limit_bytes=64 * 1024 * 1024,
        ),
    )(xflat, w1s, fb1, w2s, fb2, w3s, fb3,
      w4s, fb4, w5s, fb5, wfa, wfb, fcb)

    return out.reshape(B, _N_OUT)


# speedup vs baseline: 1.0782x; 1.0782x over previous
import jax
import jax.numpy as jnp
from jax.experimental import pallas as pl
from jax.experimental.pallas import tpu as pltpu

_BN_EPS = 1e-5
_N_OUT = 2

_L = (
    (3, 3, 16, 74, 72, 36),
    (3, 16, 32, 38, 36, 18),
    (3, 32, 64, 20, 18, 9),
    (2, 64, 128, 11, 10, 5),
    (2, 128, 32, 7, 6, 3),
)


def _rows(hp, ho):
    return (ho - 1) * hp + ho


def _body(p_ref, w1_ref, b1_ref, w2_ref, b2_ref, w3_ref, b3_ref,
          w4_ref, b4_ref, w5_ref, b5_ref, wfa_ref, wfb_ref, fcb_ref,
          out_ref, in1, act1, in2, act2, in3, act3, in4, act4, in5, act5, pool5):
    bf16 = jnp.bfloat16

    def leaky(v):
        return jnp.where(v > 0, v, 0.01 * v)

    def conv(in_ref, w_ref, b_ref, act_ref, k, hp, r):
        acc = None
        for di in range(k):
            for dj in range(k):
                part = jnp.dot(in_ref[pl.ds(di * hp + dj, r), :],
                               w_ref[di * k + dj],
                               preferred_element_type=jnp.float32)
                acc = part if acc is None else acc + part
        act_ref[...] = leaky(acc + b_ref[...])

    def pool2x2(act_ref, sp, ho2, i2):
        s = 2 * i2 * sp
        a00 = act_ref[pl.ds(s, ho2, stride=2), :]
        a01 = act_ref[pl.ds(s + 1, ho2, stride=2), :]
        a10 = act_ref[pl.ds(s + sp, ho2, stride=2), :]
        a11 = act_ref[pl.ds(s + sp + 1, ho2, stride=2), :]
        return jnp.maximum(jnp.maximum(a00, a01), jnp.maximum(a10, a11))

    def pool_pad(act_ref, sp, ho, ho2, dst_ref, dst_hp):
        zrow = jnp.zeros((1, dst_ref.shape[1]), bf16)
        for i2 in range(ho2):
            hm = pool2x2(act_ref, sp, ho2, i2).astype(bf16)
            row = jnp.concatenate([zrow, hm, zrow], axis=0)
            dst_ref[pl.ds((i2 + 1) * dst_hp, dst_hp), :] = row

    def zero_tb(dst_ref, hp):
        z = jnp.zeros((hp, dst_ref.shape[1]), bf16)
        dst_ref[pl.ds(0, hp), :] = z
        dst_ref[pl.ds((hp - 1) * hp, hp), :] = z

    hp1, ho1 = _L[0][3], _L[0][4]
    r1 = _rows(hp1, ho1)
    in1[...] = jnp.swapaxes(p_ref[0], 0, 1).astype(bf16)
    acc1 = None
    for di in range(3):
        for dj in range(3):
            part = jnp.dot(in1[pl.ds(di * hp1 + dj, r1), :],
                           w1_ref[di * 3 + dj],
                           preferred_element_type=jnp.float32)
            acc1 = part if acc1 is None else acc1 + part
    act1[...] = leaky(acc1 + b1_ref[...])

    ins = (None, in2, in3, in4, in5)
    acts = (act1, act2, act3, act4, act5)
    ws = (None, w2_ref, w3_ref, w4_ref, w5_ref)
    bs = (None, b2_ref, b3_ref, b4_ref, b5_ref)
    sps = tuple(l[3] for l in _L)

    for i in range(5):
        k, _, _, hp, ho, ho2 = _L[i]
        if i > 0:
            conv(ins[i], ws[i], bs[i], acts[i], k, hp, _rows(hp, ho))
        if i < 4:
            nhp = _L[i + 1][3]
            zero_tb(ins[i + 1], nhp)
            pool_pad(acts[i], sps[i], ho, ho2, ins[i + 1], nhp)
        else:
            for i2 in range(ho2):
                pool5[pl.ds(i2 * ho2, ho2), :] = (
                    pool2x2(acts[i], sps[i], ho2, i2).astype(bf16))

    v = pool5[...].astype(jnp.float32)
    s0 = jnp.sum(v * wfa_ref[...])
    s1 = jnp.sum(v * wfb_ref[...])
    idx = jax.lax.broadcasted_iota(jnp.int32, (1, _N_OUT), 1)
    logits = fcb_ref[...] + jnp.where(idx == 0, s0, s1)
    out_ref[...] = jax.nn.sigmoid(logits).reshape(1, 1, _N_OUT)


def kernel(c1_w, c1_b, c1_g, c1_beta, c1_m, c1_v,
           c2_w, c2_b, c2_g, c2_beta, c2_m, c2_v,
           c3_w, c3_b, c3_g, c3_beta, c3_m, c3_v,
           c4_w, c4_b, c4_g, c4_beta, c4_m, c4_v,
           c5_w, c5_b, c5_g, c5_beta, c5_m, c5_v,
           fc_w, fc_b, x):
    B = x.shape[0]
    bf16 = jnp.bfloat16

    def fold(w, b, g, beta, m, v):
        s = g * jax.lax.rsqrt(v + _BN_EPS)
        return w * s, ((b - m) * s + beta).reshape(1, -1)

    fw1, fb1 = fold(c1_w, c1_b, c1_g, c1_beta, c1_m, c1_v)
    fw2, fb2 = fold(c2_w, c2_b, c2_g, c2_beta, c2_m, c2_v)
    fw3, fb3 = fold(c3_w, c3_b, c3_g, c3_beta, c3_m, c3_v)
    fw4, fb4 = fold(c4_w, c4_b, c4_g, c4_beta, c4_m, c4_v)
    fw5, fb5 = fold(c5_w, c5_b, c5_g, c5_beta, c5_m, c5_v)

    hp1 = _L[0][3]
    xp = jnp.pad(x, ((0, 0), (0, 0), (1, 1), (1, 1)))
    xflat = xp.reshape(B, _L[0][1], hp1 * hp1)

    w1s = fw1.reshape(9, _L[0][1], _L[0][2]).astype(bf16)
    w2s = fw2.reshape(9, _L[1][1], _L[1][2]).astype(bf16)
    w3s = fw3.reshape(9, _L[2][1], _L[2][2]).astype(bf16)
    w4s = fw4.reshape(4, _L[3][1], _L[3][2]).astype(bf16)
    w5s = fw5.reshape(4, _L[4][1], _L[4][2]).astype(bf16)

    fs = _L[4][5] ** 2
    wf = fc_w.reshape(_L[4][2], fs, _N_OUT)
    wfa = wf[:, :, 0].T
    wfb = wf[:, :, 1].T
    fcb = fc_b.reshape(1, _N_OUT)

    const2 = lambda shape: pl.BlockSpec(shape, lambda b: (0, 0))
    const3 = lambda shape: pl.BlockSpec(shape, lambda b: (0, 0, 0))

    out = pl.pallas_call(
        _body,
        out_shape=jax.ShapeDtypeStruct((B, 1, _N_OUT), jnp.float32),
        grid=(B,),
        in_specs=[
            pl.BlockSpec((1, _L[0][1], hp1 * hp1), lambda b: (b, 0, 0)),
            const3(w1s.shape), const2(fb1.shape),
            const3(w2s.shape), const2(fb2.shape),
            const3(w3s.shape), const2(fb3.shape),
            const3(w4s.shape), const2(fb4.shape),
            const3(w5s.shape), const2(fb5.shape),
            const2(wfa.shape), const2(wfb.shape), const2(fcb.shape),
        ],
        out_specs=pl.BlockSpec((1, 1, _N_OUT), lambda b: (b, 0, 0)),
        scratch_shapes=[
            pltpu.VMEM((_L[0][3] * _L[0][3], _L[0][1]), bf16),
            pltpu.VMEM((_rows(_L[0][3], _L[0][4]), _L[0][2]), jnp.float32),
            pltpu.VMEM((_L[1][3] * _L[1][3], _L[1][1]), bf16),
            pltpu.VMEM((_rows(_L[1][3], _L[1][4]), _L[1][2]), jnp.float32),
            pltpu.VMEM((_L[2][3] * _L[2][3], _L[2][1]), bf16),
            pltpu.VMEM((_rows(_L[2][3], _L[2][4]), _L[2][2]), jnp.float32),
            pltpu.VMEM((_L[3][3] * _L[3][3], _L[3][1]), bf16),
            pltpu.VMEM((_rows(_L[3][3], _L[3][4]), _L[3][2]), jnp.float32),
            pltpu.VMEM((_L[4][3] * _L[4][3], _L[4][1]), bf16),
            pltpu.VMEM((_rows(_L[4][3], _L[4][4]), _L[4][2]), jnp.float32),
            pltpu.VMEM((fs, _L[4][2]), bf16),
        ],
        compiler_params=pltpu.CompilerParams(
            dimension_semantics=("parallel",),
            vmem_limit_bytes=64 * 1024 * 1024,
        ),
    )(xflat, w1s, fb1, w2s, fb2, w3s, fb3,
      w4s, fb4, w5s, fb5, wfa, wfb, fcb)

    return out.reshape(B, _N_OUT)


# speedup vs baseline: 5.0103x; 4.6469x over previous
import jax
import jax.numpy as jnp
from jax.experimental import pallas as pl
from jax.experimental.pallas import tpu as pltpu

_BN_EPS = 1e-5
_N_OUT = 2
_G = 8

_L = (
    (3, 3, 16, 74, 72, 36),
    (3, 16, 32, 38, 36, 18),
    (3, 32, 64, 20, 18, 9),
    (2, 64, 128, 11, 10, 5),
    (2, 128, 32, 7, 6, 3),
)


def _rows(hp, ho):
    return (ho - 1) * hp + ho


def _body(x_ref, w1_ref, b1_ref, w2_ref, b2_ref, w3_ref, b3_ref,
          w4_ref, b4_ref, w5_ref, b5_ref, wfa_ref, wfb_ref, gsel_ref, fcb_ref,
          out_ref, in1, act1, in2, act2, in3, act3, in4, act4, in5, act5,
          pool5):
    bf16 = jnp.bfloat16

    def leaky(v):
        return jnp.where(v > 0, v, 0.01 * v)

    def conv(in_ref, w_ref, b_ref, act_ref, k, hp, r):
        acc = None
        for di in range(k):
            for dj in range(k):
                part = jnp.dot(in_ref[pl.ds(di * hp + dj, r), :],
                               w_ref[di * k + dj],
                               preferred_element_type=jnp.float32)
                acc = part if acc is None else acc + part
        a = leaky(acc + b_ref[...])
        for t in range(act_ref.shape[0]):
            act_ref[t] = a[:, 128 * t:128 * (t + 1)]

    def pool2x2(act_ref, t, sp, ho2, i2):
        s = 2 * i2 * sp
        a00 = act_ref[t, pl.ds(s, ho2, stride=2), :]
        a01 = act_ref[t, pl.ds(s + 1, ho2, stride=2), :]
        a10 = act_ref[t, pl.ds(s + sp, ho2, stride=2), :]
        a11 = act_ref[t, pl.ds(s + sp + 1, ho2, stride=2), :]
        return jnp.maximum(jnp.maximum(a00, a01), jnp.maximum(a10, a11))

    def pool_pad(act_ref, sp, ho2, dst_ref, dst_hp):
        zrow = jnp.zeros((1, 128), bf16)
        for i2 in range(ho2):
            for t in range(act_ref.shape[0]):
                hm = pool2x2(act_ref, t, sp, ho2, i2).astype(bf16)
                row = jnp.concatenate([zrow, hm, zrow], axis=0)
                dst_ref[pl.ds((i2 + 1) * dst_hp, dst_hp),
                        128 * t:128 * (t + 1)] = row

    def zero_tb(dst_ref, hp):
        z = jnp.zeros((hp, dst_ref.shape[1]), bf16)
        dst_ref[pl.ds(0, hp), :] = z
        dst_ref[pl.ds((hp - 1) * hp, hp), :] = z

    in1[...] = jnp.swapaxes(x_ref[0], 0, 1).astype(bf16)

    ins = (in1, in2, in3, in4, in5)
    acts = (act1, act2, act3, act4, act5)
    ws = (w1_ref, w2_ref, w3_ref, w4_ref, w5_ref)
    bs = (b1_ref, b2_ref, b3_ref, b4_ref, b5_ref)
    sps = tuple(l[3] for l in _L)

    for i in range(5):
        k, _, _, hp, ho, ho2 = _L[i]
        conv(ins[i], ws[i], bs[i], acts[i], k, hp, _rows(hp, ho))
        if i < 4:
            nhp = _L[i + 1][3]
            zero_tb(ins[i + 1], nhp)
            pool_pad(acts[i], sps[i], ho2, ins[i + 1], nhp)
        else:
            for i2 in range(ho2):
                for t in range(acts[i].shape[0]):
                    pool5[pl.ds(i2 * ho2, ho2), 128 * t:128 * (t + 1)] = (
                        pool2x2(acts[i], t, sps[i], ho2, i2).astype(bf16))

    v = pool5[...].astype(jnp.float32)
    cs0 = jnp.sum(v * wfa_ref[...], axis=0, keepdims=True)
    cs1 = jnp.sum(v * wfb_ref[...], axis=0, keepdims=True)
    cs = jnp.concatenate([cs0, cs1], axis=0)
    logits = jnp.dot(cs, gsel_ref[...],
                     preferred_element_type=jnp.float32) + fcb_ref[...]
    out_ref[...] = jax.nn.sigmoid(logits)[None]


def kernel(c1_w, c1_b, c1_g, c1_beta, c1_m, c1_v,
           c2_w, c2_b, c2_g, c2_beta, c2_m, c2_v,
           c3_w, c3_b, c3_g, c3_beta, c3_m, c3_v,
           c4_w, c4_b, c4_g, c4_beta, c4_m, c4_v,
           c5_w, c5_b, c5_g, c5_beta, c5_m, c5_v,
           fc_w, fc_b, x):
    B = x.shape[0]
    bf16 = jnp.bfloat16

    def fold(w, b, g, beta, m, v):
        s = g * jax.lax.rsqrt(v + _BN_EPS)
        return w * s, ((b - m) * s + beta).reshape(1, -1)

    fw1, fb1 = fold(c1_w, c1_b, c1_g, c1_beta, c1_m, c1_v)
    fw2, fb2 = fold(c2_w, c2_b, c2_g, c2_beta, c2_m, c2_v)
    fw3, fb3 = fold(c3_w, c3_b, c3_g, c3_beta, c3_m, c3_v)
    fw4, fb4 = fold(c4_w, c4_b, c4_g, c4_beta, c4_m, c4_v)
    fw5, fb5 = fold(c5_w, c5_b, c5_g, c5_beta, c5_m, c5_v)

    def blockdiag(w, ksz, cin, cout):
        slab = w.reshape(ksz * ksz, cin, cout)
        out = jnp.zeros((ksz * ksz, _G * cin, _G * cout), jnp.float32)
        for g in range(_G):
            out = out.at[:, g * cin:(g + 1) * cin,
                         g * cout:(g + 1) * cout].set(slab)
        return out.astype(bf16)

    w1s = blockdiag(fw1, 3, _L[0][1], _L[0][2])
    w2s = blockdiag(fw2, 3, _L[1][1], _L[1][2])
    w3s = blockdiag(fw3, 3, _L[2][1], _L[2][2])
    w4s = blockdiag(fw4, 2, _L[3][1], _L[3][2])
    w5s = blockdiag(fw5, 2, _L[4][1], _L[4][2])
    tb1 = jnp.tile(fb1, (1, _G))
    tb2 = jnp.tile(fb2, (1, _G))
    tb3 = jnp.tile(fb3, (1, _G))
    tb4 = jnp.tile(fb4, (1, _G))
    tb5 = jnp.tile(fb5, (1, _G))

    hp1 = _L[0][3]
    Bp = -(-B // _G) * _G
    xq = jnp.pad(x, ((0, Bp - B), (0, 0), (1, 1), (1, 1)))
    xflat = xq.reshape(Bp // _G, _G * _L[0][1], hp1 * hp1)

    fs = _L[4][5] ** 2
    c5o = _L[4][2]
    wf = fc_w.reshape(c5o, fs, _N_OUT)
    wfa = jnp.tile(wf[:, :, 0].T, (1, _G))
    wfb = jnp.tile(wf[:, :, 1].T, (1, _G))
    gsel = jnp.repeat(jnp.eye(_G, dtype=jnp.float32), c5o, axis=0)
    fcb = jnp.tile(fc_b.reshape(_N_OUT, 1), (1, _G))

    const2 = lambda shape: pl.BlockSpec(shape, lambda b: (0, 0))
    const3 = lambda shape: pl.BlockSpec(shape, lambda b: (0, 0, 0))

    out = pl.pallas_call(
        _body,
        out_shape=jax.ShapeDtypeStruct((Bp // _G, _N_OUT, _G), jnp.float32),
        grid=(Bp // _G,),
        in_specs=[
            pl.BlockSpec((1, _G * _L[0][1], hp1 * hp1), lambda b: (b, 0, 0)),
            const3(w1s.shape), const2(tb1.shape),
            const3(w2s.shape), const2(tb2.shape),
            const3(w3s.shape), const2(tb3.shape),
            const3(w4s.shape), const2(tb4.shape),
            const3(w5s.shape), const2(tb5.shape),
            const2(wfa.shape), const2(wfb.shape),
            const2(gsel.shape), const2(fcb.shape),
        ],
        out_specs=pl.BlockSpec((1, _N_OUT, _G), lambda b: (b, 0, 0)),
        scratch_shapes=[
            pltpu.VMEM((_L[0][3] ** 2, _G * _L[0][1]), bf16),
            pltpu.VMEM((_G * _L[0][2] // 128, _rows(_L[0][3], _L[0][4]), 128),
                       jnp.float32),
            pltpu.VMEM((_L[1][3] ** 2, _G * _L[1][1]), bf16),
            pltpu.VMEM((_G * _L[1][2] // 128, _rows(_L[1][3], _L[1][4]), 128),
                       jnp.float32),
            pltpu.VMEM((_L[2][3] ** 2, _G * _L[2][1]), bf16),
            pltpu.VMEM((_G * _L[2][2] // 128, _rows(_L[2][3], _L[2][4]), 128),
                       jnp.float32),
            pltpu.VMEM((_L[3][3] ** 2, _G * _L[3][1]), bf16),
            pltpu.VMEM((_G * _L[3][2] // 128, _rows(_L[3][3], _L[3][4]), 128),
                       jnp.float32),
            pltpu.VMEM((_L[4][3] ** 2, _G * _L[4][1]), bf16),
            pltpu.VMEM((_G * _L[4][2] // 128, _rows(_L[4][3], _L[4][4]), 128),
                       jnp.float32),
            pltpu.VMEM((fs, _G * _L[4][2]), bf16),
        ],
        compiler_params=pltpu.CompilerParams(
            dimension_semantics=("parallel",),
            vmem_limit_bytes=100 * 1024 * 1024,
        ),
    )(xflat, w1s, tb1, w2s, tb2, w3s, tb3,
      w4s, tb4, w5s, tb5, wfa, wfb, gsel, fcb)

    return jnp.transpose(out, (0, 2, 1)).reshape(Bp, _N_OUT)[:B]


# speedup vs baseline: 6.5043x; 1.2982x over previous
import jax
import jax.numpy as jnp
from jax.experimental import pallas as pl
from jax.experimental.pallas import tpu as pltpu

_BN_EPS = 1e-5
_N_OUT = 2
_G = 8
_W1 = 80

_L = (
    (3, 3, 16, 74, 72, 36),
    (3, 16, 32, 38, 36, 18),
    (3, 32, 64, 20, 18, 9),
    (2, 64, 128, 11, 10, 5),
    (2, 128, 32, 7, 6, 3),
)


def _rows(hp, ho):
    return (ho - 1) * hp + ho


def _body(x_ref, w1_ref, b1_ref, w2_ref, b2_ref, w3_ref, b3_ref,
          w4_ref, b4_ref, w5_ref, b5_ref, wfa_ref, wfb_ref, gsel_ref, fcb_ref,
          out_ref, in1, act1, in2, act2, in3, act3, in4, act4, in5, act5,
          pool5):
    bf16 = jnp.bfloat16

    def leaky(v):
        return jnp.where(v > 0, v, 0.01 * v)

    def conv(in_ref, w_ref, b_ref, act_ref, k, hp, r):
        acc = None
        for di in range(k):
            for dj in range(k):
                part = jnp.dot(in_ref[pl.ds(di * hp + dj, r), :],
                               w_ref[di * k + dj],
                               preferred_element_type=jnp.float32)
                acc = part if acc is None else acc + part
        a = leaky(acc + b_ref[...])
        for t in range(act_ref.shape[0]):
            act_ref[t] = a[:, 128 * t:128 * (t + 1)]

    def pool2x2(act_ref, t, sp, ho2, i2):
        s = 2 * i2 * sp
        a00 = act_ref[t, pl.ds(s, ho2, stride=2), :]
        a01 = act_ref[t, pl.ds(s + 1, ho2, stride=2), :]
        a10 = act_ref[t, pl.ds(s + sp, ho2, stride=2), :]
        a11 = act_ref[t, pl.ds(s + sp + 1, ho2, stride=2), :]
        return jnp.maximum(jnp.maximum(a00, a01), jnp.maximum(a10, a11))

    def pool_pad(act_ref, sp, ho2, dst_ref, dst_hp):
        zrow = jnp.zeros((1, 128), bf16)
        for i2 in range(ho2):
            for t in range(act_ref.shape[0]):
                hm = pool2x2(act_ref, t, sp, ho2, i2).astype(bf16)
                row = jnp.concatenate([zrow, hm, zrow], axis=0)
                dst_ref[pl.ds((i2 + 1) * dst_hp, dst_hp),
                        128 * t:128 * (t + 1)] = row

    def zero_tb(dst_ref, hp):
        z = jnp.zeros((hp, dst_ref.shape[1]), bf16)
        dst_ref[pl.ds(0, hp), :] = z
        dst_ref[pl.ds((hp - 1) * hp, hp), :] = z

    xcm = x_ref[0]
    lr = _W1 * _L[0][3] - 2
    xsh = jnp.concatenate([xcm[:, 0:lr], xcm[:, 1:lr + 1], xcm[:, 2:lr + 2]],
                          axis=0)
    in1[pl.ds(0, lr), :] = jnp.swapaxes(xsh, 0, 1).astype(bf16)

    r1 = (_L[0][4] - 1) * _W1 + _L[0][4]
    acc1 = None
    for di in range(3):
        part = jnp.dot(in1[pl.ds(di * _W1, r1), :], w1_ref[di],
                       preferred_element_type=jnp.float32)
        acc1 = part if acc1 is None else acc1 + part
    act1[0] = leaky(acc1 + b1_ref[...])

    ins = (None, in2, in3, in4, in5)
    acts = (act1, act2, act3, act4, act5)
    ws = (None, w2_ref, w3_ref, w4_ref, w5_ref)
    bs = (None, b2_ref, b3_ref, b4_ref, b5_ref)
    sps = (_W1,) + tuple(l[3] for l in _L[1:])

    for i in range(5):
        k, _, _, hp, ho, ho2 = _L[i]
        if i > 0:
            conv(ins[i], ws[i], bs[i], acts[i], k, hp, _rows(hp, ho))
        if i < 4:
            nhp = _L[i + 1][3]
            zero_tb(ins[i + 1], nhp)
            pool_pad(acts[i], sps[i], ho2, ins[i + 1], nhp)
        else:
            for i2 in range(ho2):
                for t in range(acts[i].shape[0]):
                    pool5[pl.ds(i2 * ho2, ho2), 128 * t:128 * (t + 1)] = (
                        pool2x2(acts[i], t, sps[i], ho2, i2).astype(bf16))

    v = pool5[...].astype(jnp.float32)
    cs0 = jnp.sum(v * wfa_ref[...], axis=0, keepdims=True)
    cs1 = jnp.sum(v * wfb_ref[...], axis=0, keepdims=True)
    cs = jnp.concatenate([cs0, cs1], axis=0)
    logits = jnp.dot(cs, gsel_ref[...],
                     preferred_element_type=jnp.float32) + fcb_ref[...]
    out_ref[...] = jax.nn.sigmoid(logits)[None]


def kernel(c1_w, c1_b, c1_g, c1_beta, c1_m, c1_v,
           c2_w, c2_b, c2_g, c2_beta, c2_m, c2_v,
           c3_w, c3_b, c3_g, c3_beta, c3_m, c3_v,
           c4_w, c4_b, c4_g, c4_beta, c4_m, c4_v,
           c5_w, c5_b, c5_g, c5_beta, c5_m, c5_v,
           fc_w, fc_b, x):
    B = x.shape[0]
    bf16 = jnp.bfloat16

    def fold(w, b, g, beta, m, v):
        s = g * jax.lax.rsqrt(v + _BN_EPS)
        return w * s, ((b - m) * s + beta).reshape(1, -1)

    fw1, fb1 = fold(c1_w, c1_b, c1_g, c1_beta, c1_m, c1_v)
    fw2, fb2 = fold(c2_w, c2_b, c2_g, c2_beta, c2_m, c2_v)
    fw3, fb3 = fold(c3_w, c3_b, c3_g, c3_beta, c3_m, c3_v)
    fw4, fb4 = fold(c4_w, c4_b, c4_g, c4_beta, c4_m, c4_v)
    fw5, fb5 = fold(c5_w, c5_b, c5_g, c5_beta, c5_m, c5_v)

    def blockdiag(w, ksz, cin, cout):
        slab = w.reshape(ksz * ksz, cin, cout)
        out = jnp.zeros((ksz * ksz, _G * cin, _G * cout), jnp.float32)
        for g in range(_G):
            out = out.at[:, g * cin:(g + 1) * cin,
                         g * cout:(g + 1) * cout].set(slab)
        return out.astype(bf16)

    w1p = jnp.zeros((3, 3 * _G * 3, _G * 16), jnp.float32)
    for g in range(_G):
        for dj in range(3):
            w1p = w1p.at[:, dj * 24 + g * 3:dj * 24 + g * 3 + 3,
                         g * 16:(g + 1) * 16].set(fw1[:, dj])
    w1s = w1p.astype(bf16)
    w2s = blockdiag(fw2, 3, _L[1][1], _L[1][2])
    w3s = blockdiag(fw3, 3, _L[2][1], _L[2][2])
    w4s = blockdiag(fw4, 2, _L[3][1], _L[3][2])
    w5s = blockdiag(fw5, 2, _L[4][1], _L[4][2])
    tb1 = jnp.tile(fb1, (1, _G))
    tb2 = jnp.tile(fb2, (1, _G))
    tb3 = jnp.tile(fb3, (1, _G))
    tb4 = jnp.tile(fb4, (1, _G))
    tb5 = jnp.tile(fb5, (1, _G))

    hp1 = _L[0][3]
    Bp = -(-B // _G) * _G
    xq = jnp.pad(x, ((0, Bp - B), (0, 0), (1, 1), (1, _W1 - 73)))
    xflat = xq.reshape(Bp // _G, _G * _L[0][1], hp1 * _W1)

    fs = _L[4][5] ** 2
    c5o = _L[4][2]
    wf = fc_w.reshape(c5o, fs, _N_OUT)
    wfa = jnp.tile(wf[:, :, 0].T, (1, _G))
    wfb = jnp.tile(wf[:, :, 1].T, (1, _G))
    gsel = jnp.repeat(jnp.eye(_G, dtype=jnp.float32), c5o, axis=0)
    fcb = jnp.tile(fc_b.reshape(_N_OUT, 1), (1, _G))

    const2 = lambda shape: pl.BlockSpec(shape, lambda b: (0, 0))
    const3 = lambda shape: pl.BlockSpec(shape, lambda b: (0, 0, 0))

    out = pl.pallas_call(
        _body,
        out_shape=jax.ShapeDtypeStruct((Bp // _G, _N_OUT, _G), jnp.float32),
        grid=(Bp // _G,),
        in_specs=[
            pl.BlockSpec((1, _G * _L[0][1], hp1 * _W1), lambda b: (b, 0, 0)),
            const3(w1s.shape), const2(tb1.shape),
            const3(w2s.shape), const2(tb2.shape),
            const3(w3s.shape), const2(tb3.shape),
            const3(w4s.shape), const2(tb4.shape),
            const3(w5s.shape), const2(tb5.shape),
            const2(wfa.shape), const2(wfb.shape),
            const2(gsel.shape), const2(fcb.shape),
        ],
        out_specs=pl.BlockSpec((1, _N_OUT, _G), lambda b: (b, 0, 0)),
        scratch_shapes=[
            pltpu.VMEM((_L[0][3] * _W1, 3 * _G * _L[0][1]), bf16),
            pltpu.VMEM((1, (_L[0][4] - 1) * _W1 + _L[0][4], 128),
                       jnp.float32),
            pltpu.VMEM((_L[1][3] ** 2, _G * _L[1][1]), bf16),
            pltpu.VMEM((_G * _L[1][2] // 128, _rows(_L[1][3], _L[1][4]), 128),
                       jnp.float32),
            pltpu.VMEM((_L[2][3] ** 2, _G * _L[2][1]), bf16),
            pltpu.VMEM((_G * _L[2][2] // 128, _rows(_L[2][3], _L[2][4]), 128),
                       jnp.float32),
            pltpu.VMEM((_L[3][3] ** 2, _G * _L[3][1]), bf16),
            pltpu.VMEM((_G * _L[3][2] // 128, _rows(_L[3][3], _L[3][4]), 128),
                       jnp.float32),
            pltpu.VMEM((_L[4][3] ** 2, _G * _L[4][1]), bf16),
            pltpu.VMEM((_G * _L[4][2] // 128, _rows(_L[4][3], _L[4][4]), 128),
                       jnp.float32),
            pltpu.VMEM((fs, _G * _L[4][2]), bf16),
        ],
        compiler_params=pltpu.CompilerParams(
            dimension_semantics=("parallel",),
            vmem_limit_bytes=100 * 1024 * 1024,
        ),
    )(xflat, w1s, tb1, w2s, tb2, w3s, tb3,
      w4s, tb4, w5s, tb5, wfa, wfb, gsel, fcb)

    return jnp.transpose(out, (0, 2, 1)).reshape(Bp, _N_OUT)[:B]
